# baseline (speedup 1.0000x reference)
import jax
import jax.numpy as jnp
from jax.experimental import pallas as pl


def _mlp_body(p_ref, w1_ref, b1_ref, w2_ref, b2_ref, o_ref):
    h = jax.nn.relu(p_ref[...] @ w1_ref[...] + b1_ref[...][None, :])
    o_ref[...] = h @ w2_ref[...] + b2_ref[...][None, :]


def kernel(x, edge_index, batch, W, att_src, att_dst, bias, W1, b1, W2, b2):
    src = edge_index[0]
    dst = edge_index[1]
    N = x.shape[0]
    G = 64
    xh = jnp.einsum('nf,fhc->nhc', x, W)
    a_src = jnp.sum(xh * att_src[None, :, :], axis=-1)
    a_dst = jnp.sum(xh * att_dst[None, :, :], axis=-1)
    alpha = a_src[src] + a_dst[dst]
    alpha = jax.nn.leaky_relu(alpha, negative_slope=0.2)
    amax = jax.ops.segment_max(alpha, dst, num_segments=N)
    amax = jnp.where(jnp.isfinite(amax), amax, 0.0)
    alpha = jnp.exp(alpha - jax.lax.stop_gradient(amax)[dst])
    asum = jax.ops.segment_sum(alpha, dst, num_segments=N)
    alpha = alpha / (asum[dst] + 1e-16)
    msg = xh[src] * alpha[:, :, None]
    agg = jax.ops.segment_sum(msg, dst, num_segments=N)
    out = jnp.mean(agg, axis=1) + bias[None, :]
    out = jax.nn.relu(out)
    sums = jax.ops.segment_sum(out, batch, num_segments=G)
    cnts = jax.ops.segment_sum(jnp.ones((N, 1), jnp.float32), batch, num_segments=G)
    pooled = sums / jnp.maximum(cnts, 1.0)

    return pl.pallas_call(
        _mlp_body,
        out_shape=jax.ShapeDtypeStruct((G, W2.shape[1]), jnp.float32),
    )(pooled, W1, b1, W2, b2)


# speedup vs baseline: 27.6760x; 27.6760x over previous
import jax
import jax.numpy as jnp
from jax import lax
from jax.experimental import pallas as pl
from jax.experimental.pallas import tpu as pltpu
from jax.experimental.pallas import tpu_sc as plsc

N = 10000
E = 160000
F = 256
H = 2
C = 512
G = 64

NC = 2
NS = 16
L = 16
NW = NC * NS

EPT = 5008
E_PAD = EPT * NW
NB = 10
BKT = 1024
LIST = 5664
ACC_ROWS = 1152
MSGW = 640
K = 32
A_ROWS = 10304
AGG_ROWS = NB * BKT
SENTINEL = 1 << 20


def _attn_body(x_ref, w2d_ref, as_ref, ad_ref, o_ref):
    w2d = w2d_ref[...]
    a_s = as_ref[...]
    a_d = ad_ref[...]
    dn = (((1,), (1,)), ((), ()))
    cols = []
    for h in range(H):
        cols.append(lax.dot_general(w2d[:, h * C:(h + 1) * C], a_s[h:h + 1, :], dn))
    for h in range(H):
        cols.append(lax.dot_general(w2d[:, h * C:(h + 1) * C], a_d[h:h + 1, :], dn))
    wf = jnp.concatenate(cols + [jnp.zeros((F, 12), jnp.float32)], axis=1)
    o_ref[...] = x_ref[...] @ wf


def _attn_call(x, w2d, att_src, att_dst):
    return pl.pallas_call(
        _attn_body,
        out_shape=jax.ShapeDtypeStruct((N, 16), jnp.float32),
    )(x, w2d, att_src, att_dst)


def _sc_edge_body(src_hbm, dst_hbm, a_hbm, x_hbm, aggx_hbm,
                  src_sl, dst_sl, src_l, dstg_l, xbuf, msgbuf, sabuf, dabuf,
                  wbuf, idxbuf, zerobuf, flushbuf, acc, cnt_sm, start_sm,
                  sem0, sem1, sem2):
    c = lax.axis_index("c")
    s = lax.axis_index("s")
    wid = s * NC + c
    t0 = pl.multiple_of(wid * EPT, 16)
    iota = lax.iota(jnp.int32, L)

    pltpu.sync_copy(src_hbm.at[pl.ds(t0, EPT)], src_sl)
    pltpu.sync_copy(dst_hbm.at[pl.ds(t0, EPT)], dst_sl)

    @pl.loop(0, 16)
    def _(r):
        @pl.loop(0, MSGW // L)
        def _(q):
            zerobuf[r, pl.ds(q * L, L)] = jnp.zeros((L,), jnp.float32)

    @pl.loop(0, K)
    def _(r):
        @pl.loop(0, (MSGW - 2 * F) // L)
        def _(q):
            msgbuf[r, pl.ds(2 * F + q * L, L)] = jnp.zeros((L,), jnp.float32)

    def count_a(i, cnts):
        dvec = dst_sl[pl.ds(i * L, L)]
        new = []
        for b in range(NB):
            m = (dvec >= b * BKT) & (dvec < (b + 1) * BKT)
            new.append(cnts[b] + plsc.all_reduce_population_count(m))
        return tuple(new)

    z = jnp.zeros((L,), jnp.int32)
    cnts = lax.fori_loop(0, EPT // L, count_a, (z,) * NB)
    cnt_sc = [jnp.max(v) for v in cnts]

    starts = []
    nxt = jnp.int32(0)
    for b in range(NB):
        starts.append(nxt)
        nxt = nxt + ((cnt_sc[b] + 63) // 32) * 32

    def part_a(i, cnts2):
        svec = src_sl[pl.ds(i * L, L)]
        dvec = dst_sl[pl.ds(i * L, L)]
        new = []
        for b in range(NB):
            m = (dvec >= b * BKT) & (dvec < (b + 1) * BKT)
            mi = jnp.where(m, 1, 0)
            pos = plsc.cumsum(mi) - 1 + cnts2[b] + starts[b]
            plsc.store_scatter(src_l, [pos], svec, mask=m)
            plsc.store_scatter(dstg_l, [pos], dvec, mask=m)
            new.append(cnts2[b] + plsc.all_reduce_population_count(m))
        return tuple(new)

    lax.fori_loop(0, EPT // L, part_a, (z,) * NB)

    iota32_0 = lax.iota(jnp.int32, L)
    for b in range(NB):
        for half in range(2):
            pos = starts[b] + cnt_sc[b] + half * L + iota32_0
            plsc.store_scatter(src_l, [pos], jnp.zeros((L,), jnp.int32))
            plsc.store_scatter(dstg_l, [pos],
                               jnp.full((L,), b * BKT + BKT, jnp.int32))

    for b in range(NB):
        cnt_sm[b] = cnt_sc[b]
        start_sm[b] = starts[b]

    rows_per = ACC_ROWS // NS

    def round_body(r, rcarry):
        b_dyn = r
        lo = r * BKT
        lbase = start_sm[r]

        plsc.subcore_barrier()
        zbase = pl.multiple_of(s * rows_per, 8)
        @pl.loop(0, rows_per // 16)
        def _(k2):
            pltpu.sync_copy(zerobuf, acc.at[pl.ds(zbase + k2 * 16, 16)])
        pltpu.sync_copy(zerobuf.at[pl.ds(0, rows_per % 16)],
                        acc.at[pl.ds(zbase + (rows_per // 16) * 16, rows_per % 16)])
        plsc.subcore_barrier()

        cnt = cnt_sm[r]
        nb = (cnt + (K - 1)) // K

        def blk(j, carry):
            base = pl.multiple_of(lbase + j * K, K)
            cp_x = pltpu.async_copy(x_hbm.at[src_l.at[pl.ds(base, K)]], xbuf, sem0)
            cp_a = pltpu.async_copy(a_hbm.at[src_l.at[pl.ds(base, K)]], sabuf, sem1)
            cp_d = pltpu.async_copy(a_hbm.at[dstg_l.at[pl.ds(base, K)]], dabuf, sem2)
            for hh in range(K // L):
                dg = dstg_l[pl.ds(base + hh * L, L)]
                idxbuf[0, pl.ds(hh * L, L)] = dg - lo
            cp_a.wait()
            cp_d.wait()
            for hh in range(K // L):
                rows = iota + hh * L
                a_s0 = plsc.load_gather(sabuf, [rows, jnp.full((L,), 0, jnp.int32)])
                a_s1 = plsc.load_gather(sabuf, [rows, jnp.full((L,), 1, jnp.int32)])
                a_d0 = plsc.load_gather(dabuf, [rows, jnp.full((L,), 2, jnp.int32)])
                a_d1 = plsc.load_gather(dabuf, [rows, jnp.full((L,), 3, jnp.int32)])
                l0 = a_s0 + a_d0
                l1 = a_s1 + a_d1
                w0 = jnp.exp(jnp.maximum(l0, 0.2 * l0))
                w1 = jnp.exp(jnp.maximum(l1, 0.2 * l1))
                wbuf[0, pl.ds(hh * L, L)] = w0
                wbuf[1, pl.ds(hh * L, L)] = w1
            cp_x.wait()

            for hh in range(K // L):
                wv0 = wbuf[0, pl.ds(hh * L, L)]
                wv1 = wbuf[1, pl.ds(hh * L, L)]
                for jj in range(L):
                    j2 = hh * L + jj
                    w0s = wv0[jj]
                    w1s = wv1[jj]

                    @pl.loop(0, F // L)
                    def _(f):
                        v = xbuf[j2, pl.ds(f * L, L)]
                        msgbuf[j2, pl.ds(f * L, L)] = v * w0s
                        msgbuf[j2, pl.ds(F + f * L, L)] = v * w1s

                    wv = jnp.where(iota == 0, w0s,
                                   jnp.where(iota == 1, w1s, 0.0))
                    msgbuf[j2, pl.ds(2 * F, L)] = wv

            pltpu.sync_copy(msgbuf, acc.at[idxbuf.at[0]], add=True)
            return carry

        lax.fori_loop(0, nb, blk, 0)

        plsc.subcore_barrier()
        fbase = pl.multiple_of(s * (BKT // NS), 16)
        @pl.loop(0, (BKT // NS) // 16)
        def _(k3):
            pltpu.sync_copy(acc.at[pl.ds(fbase + k3 * 16, 16)],
                            flushbuf.at[pl.ds(0, 16)])
            pltpu.sync_copy(
                flushbuf.at[pl.ds(0, 16)],
                aggx_hbm.at[c, pl.ds(
                    pl.multiple_of(b_dyn * BKT + fbase + k3 * 16, 16), 16)])
        return rcarry

    lax.fori_loop(0, NB, round_body, 0)


def _sc_edge_call(src_p, dst_p, a_pad, x):
    mesh = plsc.VectorSubcoreMesh(core_axis_name="c", subcore_axis_name="s")
    f32 = jnp.float32
    return pl.kernel(
        _sc_edge_body,
        out_type=jax.ShapeDtypeStruct((NC, AGG_ROWS, MSGW), f32),
        mesh=mesh,
        compiler_params=pltpu.CompilerParams(
            use_tc_tiling_on_sc=False, needs_layout_passes=False),
        scratch_types=[
            pltpu.VMEM((EPT,), jnp.int32),
            pltpu.VMEM((EPT,), jnp.int32),
            pltpu.VMEM((LIST,), jnp.int32),
            pltpu.VMEM((LIST,), jnp.int32),
            pltpu.VMEM((K, F), f32),
            pltpu.VMEM((K, MSGW), f32),
            pltpu.VMEM((K, 16), f32),
            pltpu.VMEM((K, 16), f32),
            pltpu.VMEM((2, K), f32),
            pltpu.VMEM((2, K), jnp.int32),
            pltpu.VMEM((16, MSGW), f32),
            pltpu.VMEM((K, MSGW), f32),
            pltpu.VMEM_SHARED((ACC_ROWS, MSGW), f32),
            pltpu.SMEM((NB,), jnp.int32),
            pltpu.SMEM((NB,), jnp.int32),
            pltpu.SemaphoreType.DMA,
            pltpu.SemaphoreType.DMA,
            pltpu.SemaphoreType.DMA,
        ],
    )(src_p, dst_p, a_pad, x)


def _tail_body(a0_ref, a1_ref, w0_ref, w1_ref, bias_ref, batch_ref,
               w1m_ref, b1_ref, w2m_ref, b2_ref, o_ref, pool_acc, cnt_acc):
    i = pl.program_id(0)
    blk = (a0_ref[...] + a1_ref[...])[0]
    ag0 = blk[:, 0:F] @ w0_ref[...]
    ag1 = blk[:, F:2 * F] @ w1_ref[...]
    s0 = blk[:, 2 * F:2 * F + 1]
    s1 = blk[:, 2 * F + 1:2 * F + 2]
    out = jnp.maximum(
        (ag0 / (s0 + 1e-16) + ag1 / (s1 + 1e-16)) * 0.5 + bias_ref[...][None, :],
        0.0)
    b = batch_ref[0, 0, :]
    P = (b[None, :] == lax.broadcasted_iota(jnp.int32, (G, 1), 0)).astype(jnp.float32)

    @pl.when(i == 0)
    def _():
        pool_acc[...] = jnp.zeros_like(pool_acc)
        cnt_acc[...] = jnp.zeros_like(cnt_acc)

    pool_acc[...] += P @ out
    cnt_acc[...] += jnp.sum(P, axis=1, keepdims=True)

    @pl.when(i == pl.num_programs(0) - 1)
    def _():
        pooled = pool_acc[...] / jnp.maximum(cnt_acc[...], 1.0)
        h1 = jnp.maximum(pooled @ w1m_ref[...] + b1_ref[...][None, :], 0.0)
        o_ref[...] = h1 @ w2m_ref[...] + b2_ref[...][None, :]


def _tail_call(aggx, w0, w1, bias, batch_p, W1, b1, W2, b2):
    nblk = AGG_ROWS // 128
    return pl.pallas_call(
        _tail_body,
        grid=(nblk,),
        in_specs=[
            pl.BlockSpec((1, 128, MSGW), lambda i: (0, i, 0)),
            pl.BlockSpec((1, 128, MSGW), lambda i: (1, i, 0)),
            pl.BlockSpec((F, C), lambda i: (0, 0)),
            pl.BlockSpec((F, C), lambda i: (0, 0)),
            pl.BlockSpec((C,), lambda i: (0,)),
            pl.BlockSpec((1, 1, 128), lambda i: (i, 0, 0)),
            pl.BlockSpec((C, C), lambda i: (0, 0)),
            pl.BlockSpec((C,), lambda i: (0,)),
            pl.BlockSpec((C, 1), lambda i: (0, 0)),
            pl.BlockSpec((1,), lambda i: (0,)),
        ],
        out_specs=pl.BlockSpec((G, 1), lambda i: (0, 0)),
        out_shape=jax.ShapeDtypeStruct((G, 1), jnp.float32),
        scratch_shapes=[
            pltpu.VMEM((G, C), jnp.float32),
            pltpu.VMEM((G, 1), jnp.float32),
        ],
    )(aggx, aggx, w0, w1, bias, batch_p, W1, b1, W2, b2)


def kernel(x, edge_index, batch, W, att_src, att_dst, bias, W1, b1, W2, b2):
    src = edge_index[0]
    dst = edge_index[1]
    pad = E_PAD - E
    src_p = jnp.concatenate([src, jnp.zeros((pad,), jnp.int32)])
    dst_p = jnp.concatenate([dst, jnp.full((pad,), SENTINEL, jnp.int32)])
    w2d = W.reshape(F, H * C)

    a_arr = _attn_call(x, w2d, att_src, att_dst)
    a_pad = jnp.concatenate(
        [a_arr, jnp.zeros((A_ROWS - N, 16), jnp.float32)], axis=0)

    aggx = _sc_edge_call(src_p, dst_p, a_pad, x)

    batch_p = jnp.concatenate(
        [batch, jnp.full((AGG_ROWS - N,), G + 63, jnp.int32)]).reshape(
            AGG_ROWS // 128, 1, 128)
    return _tail_call(aggx, w2d[:, 0:C], w2d[:, C:2 * C], bias, batch_p,
                      W1, b1, W2, b2)
